# baseline (device time: 10162 ns/iter reference)
import jax
import jax.numpy as jnp
from jax import lax
from jax.experimental import pallas as pl
from jax.experimental.pallas import tpu as pltpu

N_DEV = 4

N_XFER = 3


def kernel(q, k, v):
    s_per, d = q.shape
    scale = 1.0 / (d**0.5)

    def body(
        q_ref, k_ref, v_ref, out_ref, sendbuf, comm_ref, send_sems, recv_sems
    ):
        my = lax.axis_index("i")
        left = lax.rem(my + (N_DEV - 1), N_DEV)
        right = lax.rem(my + 1, N_DEV)
        opp = lax.rem(my + 2, N_DEV)

        sendbuf[:, :d] = k_ref[...].astype(jnp.bfloat16)
        sendbuf[:, d:] = v_ref[...].astype(jnp.bfloat16)

        barrier_sem = pltpu.get_barrier_semaphore()
        for nbr in (left, right, opp):
            pl.semaphore_signal(
                barrier_sem,
                inc=1,
                device_id=(nbr,),
                device_id_type=pl.DeviceIdType.MESH,
            )
        pl.semaphore_wait(barrier_sem, 3)

        xfers = {}
        for t, dev in ((2, opp), (0, right), (1, left)):
            x = pltpu.make_async_remote_copy(
                src_ref=sendbuf,
                dst_ref=comm_ref.at[t],
                send_sem=send_sems.at[t],
                recv_sem=recv_sems.at[t],
                device_id=(dev,),
                device_id_type=pl.DeviceIdType.MESH,
            )
            x.start()
            xfers[t] = x

        qs = (q_ref[...] * scale).astype(jnp.bfloat16)

        def block(kb, vb, l, acc):
            s = lax.dot_general(
                qs, kb, (((1,), (1,)), ((), ())),
                preferred_element_type=jnp.float32,
            )
            p = jnp.exp(s)
            l_new = l + jnp.sum(p, axis=1, keepdims=True)
            acc_new = acc + lax.dot_general(
                p.astype(jnp.bfloat16), vb, (((1,), (0,)), ((), ())),
                preferred_element_type=jnp.float32,
            )
            return l_new, acc_new

        l, acc = block(
            k_ref[...].astype(jnp.bfloat16),
            v_ref[...].astype(jnp.bfloat16),
            jnp.zeros((s_per, 1), jnp.float32),
            jnp.zeros((s_per, d), jnp.float32),
        )

        for slot in range(3):
            xfers[slot].wait_recv()
            l, acc = block(comm_ref[slot, :, :d], comm_ref[slot, :, d:], l, acc)

        out_ref[...] = acc / l

        for x in xfers.values():
            x.wait_send()

    return pl.pallas_call(
        body,
        out_shape=jax.ShapeDtypeStruct((s_per, d), jnp.float32),
        in_specs=[pl.BlockSpec(memory_space=pltpu.VMEM)] * 3,
        out_specs=pl.BlockSpec(memory_space=pltpu.VMEM),
        scratch_shapes=[
            pltpu.VMEM((s_per, 2 * d), jnp.bfloat16),
            pltpu.VMEM((3, s_per, 2 * d), jnp.bfloat16),
            pltpu.SemaphoreType.DMA((N_XFER,)),
            pltpu.SemaphoreType.DMA((N_XFER,)),
        ],
        compiler_params=pltpu.CompilerParams(collective_id=0),
    )(q, k, v)


# device time: 10064 ns/iter; 1.0097x vs baseline; 1.0097x over previous
import jax
import jax.numpy as jnp
from jax import lax
from jax.experimental import pallas as pl
from jax.experimental.pallas import tpu as pltpu

N_DEV = 4

N_XFER = 3


def kernel(q, k, v):
    s_per, d = q.shape
    scale = 1.0 / (d**0.5)

    def body(
        q_ref, k_ref, v_ref, out_ref, sendbuf, comm_ref, send_sems, recv_sems
    ):
        my = lax.axis_index("i")
        left = lax.rem(my + (N_DEV - 1), N_DEV)
        right = lax.rem(my + 1, N_DEV)
        opp = lax.rem(my + 2, N_DEV)

        barrier_sem = pltpu.get_barrier_semaphore()
        for nbr in (left, right, opp):
            pl.semaphore_signal(
                barrier_sem,
                inc=1,
                device_id=(nbr,),
                device_id_type=pl.DeviceIdType.MESH,
            )

        sendbuf[:, :d] = k_ref[...].astype(jnp.bfloat16)
        sendbuf[:, d:] = v_ref[...].astype(jnp.bfloat16)

        qs = (q_ref[...] * scale).astype(jnp.bfloat16)

        def block(kb, vb, l, acc):
            s = lax.dot_general(
                qs, kb, (((1,), (1,)), ((), ())),
                preferred_element_type=jnp.float32,
            )
            p = jnp.exp(s)
            l_new = l + jnp.sum(p, axis=1, keepdims=True)
            acc_new = acc + lax.dot_general(
                p.astype(jnp.bfloat16), vb, (((1,), (0,)), ((), ())),
                preferred_element_type=jnp.float32,
            )
            return l_new, acc_new

        l, acc = block(
            k_ref[...].astype(jnp.bfloat16),
            v_ref[...].astype(jnp.bfloat16),
            jnp.zeros((s_per, 1), jnp.float32),
            jnp.zeros((s_per, d), jnp.float32),
        )

        pl.semaphore_wait(barrier_sem, 3)

        xfers = {}
        for t, dev in ((2, opp), (0, right), (1, left)):
            x = pltpu.make_async_remote_copy(
                src_ref=sendbuf,
                dst_ref=comm_ref.at[t],
                send_sem=send_sems.at[t],
                recv_sem=recv_sems.at[t],
                device_id=(dev,),
                device_id_type=pl.DeviceIdType.MESH,
            )
            x.start()
            xfers[t] = x

        for slot in range(3):
            xfers[slot].wait_recv()
            l, acc = block(comm_ref[slot, :, :d], comm_ref[slot, :, d:], l, acc)

        out_ref[...] = acc / l

        for x in xfers.values():
            x.wait_send()

    return pl.pallas_call(
        body,
        out_shape=jax.ShapeDtypeStruct((s_per, d), jnp.float32),
        in_specs=[pl.BlockSpec(memory_space=pltpu.VMEM)] * 3,
        out_specs=pl.BlockSpec(memory_space=pltpu.VMEM),
        scratch_shapes=[
            pltpu.VMEM((s_per, 2 * d), jnp.bfloat16),
            pltpu.VMEM((3, s_per, 2 * d), jnp.bfloat16),
            pltpu.SemaphoreType.DMA((N_XFER,)),
            pltpu.SemaphoreType.DMA((N_XFER,)),
        ],
        compiler_params=pltpu.CompilerParams(collective_id=0),
    )(q, k, v)


# device time: 9389 ns/iter; 1.0823x vs baseline; 1.0719x over previous
import jax
import jax.numpy as jnp
from jax import lax
from jax.experimental import pallas as pl
from jax.experimental.pallas import tpu as pltpu

N_DEV = 4

N_XFER = 3


def kernel(q, k, v):
    s_per, d = q.shape
    scale = 1.0 / (d**0.5)

    def body(
        q_ref, k_ref, v_ref, out_ref, sendbuf, comm_ref, send_sems, recv_sems,
        entry_sems,
    ):
        my = lax.axis_index("i")
        left = lax.rem(my + (N_DEV - 1), N_DEV)
        right = lax.rem(my + 1, N_DEV)
        opp = lax.rem(my + 2, N_DEV)

        barrier_sem = pltpu.get_barrier_semaphore()
        pl.semaphore_signal(
            entry_sems.at[0], inc=1,
            device_id=(left,), device_id_type=pl.DeviceIdType.MESH,
        )
        pl.semaphore_signal(
            entry_sems.at[1], inc=1,
            device_id=(right,), device_id_type=pl.DeviceIdType.MESH,
        )
        pl.semaphore_signal(
            barrier_sem, inc=1,
            device_id=(opp,), device_id_type=pl.DeviceIdType.MESH,
        )

        sendbuf[:, :d] = k_ref[...].astype(jnp.bfloat16)
        sendbuf[:, d:] = v_ref[...].astype(jnp.bfloat16)

        xfers = []
        for t, dev, wait in (
            (0, right, lambda: pl.semaphore_wait(entry_sems.at[0], 1)),
            (1, left, lambda: pl.semaphore_wait(entry_sems.at[1], 1)),
            (2, opp, lambda: pl.semaphore_wait(barrier_sem, 1)),
        ):
            wait()
            x = pltpu.make_async_remote_copy(
                src_ref=sendbuf,
                dst_ref=comm_ref.at[t],
                send_sem=send_sems.at[t],
                recv_sem=recv_sems.at[t],
                device_id=(dev,),
                device_id_type=pl.DeviceIdType.MESH,
            )
            x.start()
            xfers.append(x)

        qs = (q_ref[...] * scale).astype(jnp.bfloat16)

        def block(kb, vb, l, acc):
            s = lax.dot_general(
                qs, kb, (((1,), (1,)), ((), ())),
                preferred_element_type=jnp.float32,
            )
            p = jnp.exp(s)
            l_new = l + jnp.sum(p, axis=1, keepdims=True)
            acc_new = acc + lax.dot_general(
                p.astype(jnp.bfloat16), vb, (((1,), (0,)), ((), ())),
                preferred_element_type=jnp.float32,
            )
            return l_new, acc_new

        l, acc = block(
            k_ref[...].astype(jnp.bfloat16),
            v_ref[...].astype(jnp.bfloat16),
            jnp.zeros((s_per, 1), jnp.float32),
            jnp.zeros((s_per, d), jnp.float32),
        )

        for t in range(N_XFER):
            xfers[t].wait_recv()
            l, acc = block(comm_ref[t, :, :d], comm_ref[t, :, d:], l, acc)

        out_ref[...] = acc / l

        for x in xfers:
            x.wait_send()

    return pl.pallas_call(
        body,
        out_shape=jax.ShapeDtypeStruct((s_per, d), jnp.float32),
        in_specs=[pl.BlockSpec(memory_space=pltpu.VMEM)] * 3,
        out_specs=pl.BlockSpec(memory_space=pltpu.VMEM),
        scratch_shapes=[
            pltpu.VMEM((s_per, 2 * d), jnp.bfloat16),
            pltpu.VMEM((3, s_per, 2 * d), jnp.bfloat16),
            pltpu.SemaphoreType.DMA((N_XFER,)),
            pltpu.SemaphoreType.DMA((N_XFER,)),
            pltpu.SemaphoreType.REGULAR((2,)),
        ],
        compiler_params=pltpu.CompilerParams(collective_id=0),
    )(q, k, v)
